# baseline (speedup 1.0000x reference)
import functools

import jax
import jax.numpy as jnp
from jax import lax
from jax.experimental import pallas as pl
from jax.experimental.pallas import tpu as pltpu
from jax.experimental.pallas import tpu_sc as plsc

NR = 8192
NZ = 2048
N_QUERY = 1000000
M = (NR * NZ) // 16

NC = 2
NS = 16
NW = NC * NS
L = 16

C = 1024
GW = 128
NCORR = 2 * GW
NCHUNK = (N_QUERY + C - 1) // C
K = (NCHUNK + NW - 1) // NW
H = (K - 1) // 2


def _make_kernel():
    mesh = plsc.VectorSubcoreMesh(core_axis_name="c", subcore_axis_name="s")

    set_scratch = [
        pltpu.VMEM((C,), jnp.float32),
        pltpu.VMEM((C,), jnp.float32),
        pltpu.VMEM((2 * C,), jnp.int32),
        pltpu.VMEM((2 * C, 16), jnp.float32),
        pltpu.VMEM((C,), jnp.float32),
        pltpu.VMEM((C,), jnp.float32),
        pltpu.VMEM((C,), jnp.float32),
        pltpu.VMEM((C,), jnp.int32),
        pltpu.VMEM((C,), jnp.float32),
        pltpu.VMEM((C,), jnp.float32),
        pltpu.VMEM((C + L,), jnp.int32),
        pltpu.VMEM((C + L,), jnp.int32),
        pltpu.VMEM((C + L,), jnp.int32),
        pltpu.VMEM((NCORR, 16), jnp.float32),
        pltpu.VMEM((NCORR, 16), jnp.float32),
        pltpu.VMEM((L, 16), jnp.float32),
        pltpu.VMEM((L, 16), jnp.float32),
        pltpu.SemaphoreType.DMA,
        pltpu.SemaphoreType.DMA,
        pltpu.SemaphoreType.DMA,
    ]

    @functools.partial(
        pl.kernel,
        mesh=mesh,
        out_type=jax.ShapeDtypeStruct((N_QUERY,), jnp.float32),
        scratch_types=set_scratch + set_scratch,
        compiler_params=pltpu.CompilerParams(use_tc_tiling_on_sc=False, needs_layout_passes=False),
    )
    def k(r_hbm, z_hbm, tab_hbm, out_hbm, *scr):
        A = scr[:20]
        B = scr[20:]
        wid = lax.axis_index("s") * NC + lax.axis_index("c")
        iota = jnp.arange(L, dtype=jnp.int32)
        tab2 = tab_hbm

        def slot_off(slot):
            cid = wid + slot * NW
            return jnp.minimum(cid * C, N_QUERY - C)

        def prefill_corr(bufs):
            cf0, cf1 = bufs[10], bufs[11]

            def body(i, carry):
                s = pl.ds(i * L, L)
                cf0[s] = iota + i * L
                cf1[s] = iota + i * L
                return carry

            lax.fori_loop(0, (C + L) // L, body, 0)

        def start_rz(bufs, slot):
            r_v, z_v, sem = bufs[0], bufs[1], bufs[17]
            off = slot_off(slot)
            pltpu.async_copy(r_hbm.at[pl.ds(off, C)], r_v, sem)
            pltpu.async_copy(z_hbm.at[pl.ds(off, C)], z_v, sem)

        def drain_rz(bufs):
            r_v, z_v, sem = bufs[0], bufs[1], bufs[17]
            pltpu.make_async_copy(r_hbm.at[pl.ds(0, C)], r_v, sem).wait()
            pltpu.make_async_copy(z_hbm.at[pl.ds(0, C)], z_v, sem).wait()

        def fire_slot(bufs):
            (r_v, z_v, ip_v, blk_v, wr_v, wz_v, _o, q_v, _t01, _t11,
             cf0, cf1, cpos, cv0, cv1, _sx0, _sx1,
             _semz, sem_g, sem_c) = bufs
            drain_rz(bufs)

            def idx_body(i, cnt):
                s = pl.ds(i * L, L)
                rr = r_v[s]
                zz = z_v[s]
                ir0 = jnp.minimum(jnp.maximum(rr.astype(jnp.int32), 0), NR - 2)
                iz0 = jnp.minimum(jnp.maximum(zz.astype(jnp.int32), 0), NZ - 2)
                wr = jnp.clip(rr - ir0.astype(jnp.float32), 0.0, 1.0)
                wz = jnp.clip(zz - iz0.astype(jnp.float32), 0.0, 1.0)
                flat = ir0 * NZ + iz0
                p0 = lax.shift_right_logical(flat, 4)
                q = jnp.bitwise_and(flat, 15)
                ip_v[pl.ds(0 * C + i * L, L)] = p0
                ip_v[pl.ds(1 * C + i * L, L)] = p0 + (NZ // 16)
                q_v[s] = q
                wr_v[s] = wr
                wz_v[s] = wz
                m = q == 15
                n = plsc.all_reduce_population_count(m)[0]
                cs = pl.ds(cnt, L)
                plsc.store_compressed(cf0.at[cs], p0 + 1, mask=m)
                plsc.store_compressed(cf1.at[cs], p0 + 1 + (NZ // 16), mask=m)
                plsc.store_compressed(cpos.at[cs], iota + i * L, mask=m)
                return cnt + n

            cnt = lax.fori_loop(0, C // L, idx_body, jnp.int32(0))

            for j in range(2 * C // GW):
                s = pl.ds(j * GW, GW)
                pltpu.async_copy(tab2.at[ip_v.at[s]], blk_v.at[s], sem_g)

            for j in range(NCORR // GW):
                s = pl.ds(j * GW, GW)
                d = pl.ds(j * GW, GW)
                pltpu.async_copy(tab2.at[cf0.at[s]], cv0.at[d], sem_c)
                pltpu.async_copy(tab2.at[cf1.at[s]], cv1.at[d], sem_c)
            return cnt

        def finish_slot(bufs, slot, cnt):
            (_r, _z, _ip, blk_v, wr_v, wz_v, o_v, q_v, t01_v, t11_v,
             cf0, cf1, cpos, cv0, cv1, sx0, sx1,
             _semz, sem_g, sem_c) = bufs
            pltpu.make_async_copy(
                tab2.at[pl.ds(0, 2 * C)], blk_v, sem_g).wait()
            for j in range(NCORR // GW):
                d = pl.ds(j * GW, GW)
                pltpu.make_async_copy(
                    tab2.at[pl.ds(0, GW)], cv0.at[d], sem_c).wait()
                pltpu.make_async_copy(
                    tab2.at[pl.ds(0, GW)], cv1.at[d], sem_c).wait()

            def ext_body(i, carry):
                s = pl.ds(i * L, L)
                row = iota + i * L
                c1 = jnp.minimum(q_v[s] + 1, 15)
                t01_v[s] = plsc.load_gather(blk_v, [row, c1])
                t11_v[s] = plsc.load_gather(blk_v, [row + C, c1])
                return carry

            lax.fori_loop(0, C // L, ext_body, 0)

            zero = jnp.zeros((L,), jnp.int32)

            def merge_fast(g, carry):
                s = pl.ds(g * L, L)
                rows = iota + g * L
                v0 = plsc.load_gather(cv0, [rows, zero])
                v1 = plsc.load_gather(cv1, [rows, zero])
                pos = cpos[s]
                m2 = iota < (cnt - g * L)
                plsc.store_scatter(t01_v, [pos], v0, mask=m2)
                plsc.store_scatter(t11_v, [pos], v1, mask=m2)
                return carry

            ngf = jnp.minimum((cnt + L - 1) // L, NCORR // L)
            lax.fori_loop(0, ngf, merge_fast, 0)

            def merge_slow(g, carry):
                s = pl.ds(g * L, L)
                pltpu.async_copy(tab2.at[cf0.at[s]], sx0, sem_c).wait()
                pltpu.async_copy(tab2.at[cf1.at[s]], sx1, sem_c).wait()
                v0 = plsc.load_gather(sx0, [iota, zero])
                v1 = plsc.load_gather(sx1, [iota, zero])
                pos = cpos[s]
                m2 = iota < (cnt - g * L)
                plsc.store_scatter(t01_v, [pos], v0, mask=m2)
                plsc.store_scatter(t11_v, [pos], v1, mask=m2)
                return carry

            lax.fori_loop(NCORR // L, (cnt + L - 1) // L, merge_slow, 0)

            def comb_body(i, carry):
                s = pl.ds(i * L, L)
                row = iota + i * L
                q = q_v[s]
                t00 = plsc.load_gather(blk_v, [row, q])
                t10 = plsc.load_gather(blk_v, [row + C, q])
                t01 = t01_v[s]
                t11 = t11_v[s]
                wr = wr_v[s]
                wz = wz_v[s]
                a = t00 * (1.0 - wr) + t10 * wr
                b2 = t01 * (1.0 - wr) + t11 * wr
                o_v[s] = a * (1.0 - wz) + b2 * wz
                return carry

            lax.fori_loop(0, C // L, comb_body, 0)
            pltpu.sync_copy(o_v, out_hbm.at[pl.ds(slot_off(slot), C)])

        prefill_corr(A)
        prefill_corr(B)
        start_rz(A, 0)
        cntA0 = fire_slot(A)
        start_rz(B, 1)

        def pair_body(h, cntA):
            cntB = fire_slot(B)
            start_rz(A, 2 * h + 2)
            finish_slot(A, 2 * h, cntA)
            cntA_new = fire_slot(A)
            start_rz(B, 2 * h + 3)
            finish_slot(B, 2 * h + 1, cntB)
            return cntA_new

        cntA_last = lax.fori_loop(0, H, pair_body, cntA0)
        finish_slot(A, 2 * H, cntA_last)
        drain_rz(B)

    return k


_sc_interp = _make_kernel()


def kernel(r, z, timetable):
    tab_blocks = timetable.reshape(M, 16)
    return _sc_interp(r, z, tab_blocks)


# speedup vs baseline: 1.0105x; 1.0105x over previous
import functools

import jax
import jax.numpy as jnp
from jax import lax
from jax.experimental import pallas as pl
from jax.experimental.pallas import tpu as pltpu
from jax.experimental.pallas import tpu_sc as plsc

NR = 8192
NZ = 2048
N_QUERY = 1000000
M = (NR * NZ) // 16

NC = 2
NS = 16
NW = NC * NS
L = 16

C = 1024
GW = 128
NCORR = 2 * GW
NCHUNK = (N_QUERY + C - 1) // C
K = (NCHUNK + NW - 1) // NW
H = (K - 1) // 2


def _make_kernel():
    mesh = plsc.VectorSubcoreMesh(core_axis_name="c", subcore_axis_name="s")

    set_scratch = [
        pltpu.VMEM((C,), jnp.float32),
        pltpu.VMEM((C,), jnp.float32),
        pltpu.VMEM((2 * C,), jnp.int32),
        pltpu.VMEM((2 * C, 16), jnp.float32),
        pltpu.VMEM((C,), jnp.float32),
        pltpu.VMEM((C,), jnp.float32),
        pltpu.VMEM((C,), jnp.float32),
        pltpu.VMEM((C,), jnp.int32),
        pltpu.VMEM((C,), jnp.float32),
        pltpu.VMEM((C,), jnp.float32),
        pltpu.VMEM((C + L,), jnp.int32),
        pltpu.VMEM((C + L,), jnp.int32),
        pltpu.VMEM((C + L,), jnp.int32),
        pltpu.VMEM((NCORR, 16), jnp.float32),
        pltpu.VMEM((NCORR, 16), jnp.float32),
        pltpu.VMEM((L, 16), jnp.float32),
        pltpu.VMEM((L, 16), jnp.float32),
        pltpu.SemaphoreType.DMA,
        pltpu.SemaphoreType.DMA,
        pltpu.SemaphoreType.DMA,
    ]

    @functools.partial(
        pl.kernel,
        mesh=mesh,
        out_type=jax.ShapeDtypeStruct((N_QUERY,), jnp.float32),
        scratch_types=set_scratch + set_scratch,
        compiler_params=pltpu.CompilerParams(use_tc_tiling_on_sc=False, needs_layout_passes=False),
    )
    def k(r_hbm, z_hbm, tab_hbm, out_hbm, *scr):
        A = scr[:20]
        B = scr[20:]
        wid = lax.axis_index("s") * NC + lax.axis_index("c")
        iota = jnp.arange(L, dtype=jnp.int32)
        tab2 = tab_hbm

        def slot_off(slot):
            cid = wid + slot * NW
            return jnp.minimum(cid * C, N_QUERY - C)

        def prefill_corr(bufs):
            cf0, cf1 = bufs[10], bufs[11]

            def body(i, carry):
                s = pl.ds(i * L, L)
                cf0[s] = iota + i * L
                cf1[s] = iota + i * L
                return carry

            lax.fori_loop(0, (C + L) // L, body, 0)

        def start_rz(bufs, slot):
            r_v, z_v, sem = bufs[0], bufs[1], bufs[17]
            off = slot_off(slot)
            pltpu.async_copy(r_hbm.at[pl.ds(off, C)], r_v, sem)
            pltpu.async_copy(z_hbm.at[pl.ds(off, C)], z_v, sem)

        def drain_rz(bufs):
            r_v, z_v, sem = bufs[0], bufs[1], bufs[17]
            pltpu.make_async_copy(r_hbm.at[pl.ds(0, C)], r_v, sem).wait()
            pltpu.make_async_copy(z_hbm.at[pl.ds(0, C)], z_v, sem).wait()

        def fire_slot(bufs):
            (r_v, z_v, ip_v, blk_v, wr_v, wz_v, _o, q_v, _t01, _t11,
             cf0, cf1, cpos, cv0, cv1, _sx0, _sx1,
             _semz, sem_g, sem_c) = bufs
            drain_rz(bufs)

            def idx_body(i, cnt):
                s = pl.ds(i * L, L)
                rr = r_v[s]
                zz = z_v[s]
                ir0 = rr.astype(jnp.int32)
                iz0 = zz.astype(jnp.int32)
                wr = rr - ir0.astype(jnp.float32)
                wz = zz - iz0.astype(jnp.float32)
                flat = ir0 * NZ + iz0
                p0 = lax.shift_right_logical(flat, 4)
                q = jnp.bitwise_and(flat, 15)
                ip_v[pl.ds(0 * C + i * L, L)] = p0
                ip_v[pl.ds(1 * C + i * L, L)] = p0 + (NZ // 16)
                q_v[s] = q
                wr_v[s] = wr
                wz_v[s] = wz
                m = q == 15
                n = plsc.all_reduce_population_count(m)[0]
                cs = pl.ds(cnt, L)
                plsc.store_compressed(cf0.at[cs], p0 + 1, mask=m)
                plsc.store_compressed(cf1.at[cs], p0 + 1 + (NZ // 16), mask=m)
                plsc.store_compressed(cpos.at[cs], iota + i * L, mask=m)
                return cnt + n

            cnt = lax.fori_loop(0, C // L, idx_body, jnp.int32(0))

            for j in range(2 * C // GW):
                s = pl.ds(j * GW, GW)
                pltpu.async_copy(tab2.at[ip_v.at[s]], blk_v.at[s], sem_g)

            for j in range(NCORR // GW):
                s = pl.ds(j * GW, GW)
                d = pl.ds(j * GW, GW)
                pltpu.async_copy(tab2.at[cf0.at[s]], cv0.at[d], sem_c)
                pltpu.async_copy(tab2.at[cf1.at[s]], cv1.at[d], sem_c)
            return cnt

        def finish_slot(bufs, slot, cnt):
            (_r, _z, _ip, blk_v, wr_v, wz_v, o_v, q_v, t01_v, t11_v,
             cf0, cf1, cpos, cv0, cv1, sx0, sx1,
             _semz, sem_g, sem_c) = bufs
            pltpu.make_async_copy(
                tab2.at[pl.ds(0, 2 * C)], blk_v, sem_g).wait()
            for j in range(NCORR // GW):
                d = pl.ds(j * GW, GW)
                pltpu.make_async_copy(
                    tab2.at[pl.ds(0, GW)], cv0.at[d], sem_c).wait()
                pltpu.make_async_copy(
                    tab2.at[pl.ds(0, GW)], cv1.at[d], sem_c).wait()

            def comb_body(i, carry):
                s = pl.ds(i * L, L)
                row = iota + i * L
                q = q_v[s]
                c1 = jnp.minimum(q + 1, 15)
                t00 = plsc.load_gather(blk_v, [row, q])
                t01 = plsc.load_gather(blk_v, [row, c1])
                t10 = plsc.load_gather(blk_v, [row + C, q])
                t11 = plsc.load_gather(blk_v, [row + C, c1])
                wr = wr_v[s]
                wz = wz_v[s]
                a = t00 * (1.0 - wr) + t10 * wr
                b2 = t01 * (1.0 - wr) + t11 * wr
                o_v[s] = a * (1.0 - wz) + b2 * wz
                return carry

            lax.fori_loop(0, C // L, comb_body, 0)

            zero = jnp.zeros((L,), jnp.int32)

            def fix_group(pos, v0, v1, m2):
                pos = jnp.where(m2, pos, 0)
                qq = plsc.load_gather(q_v, [pos])
                t00 = plsc.load_gather(blk_v, [pos, qq])
                t10 = plsc.load_gather(blk_v, [pos + C, qq])
                wr = plsc.load_gather(wr_v, [pos])
                wz = plsc.load_gather(wz_v, [pos])
                a = t00 * (1.0 - wr) + t10 * wr
                b2 = v0 * (1.0 - wr) + v1 * wr
                val = a * (1.0 - wz) + b2 * wz
                plsc.store_scatter(o_v, [pos], val, mask=m2)

            def merge_fast(g, carry):
                s = pl.ds(g * L, L)
                rows = iota + g * L
                v0 = plsc.load_gather(cv0, [rows, zero])
                v1 = plsc.load_gather(cv1, [rows, zero])
                pos = cpos[s]
                m2 = iota < (cnt - g * L)
                fix_group(pos, v0, v1, m2)
                return carry

            ngf = jnp.minimum((cnt + L - 1) // L, NCORR // L)
            lax.fori_loop(0, ngf, merge_fast, 0)

            def merge_slow(g, carry):
                s = pl.ds(g * L, L)
                pltpu.async_copy(tab2.at[cf0.at[s]], sx0, sem_c).wait()
                pltpu.async_copy(tab2.at[cf1.at[s]], sx1, sem_c).wait()
                v0 = plsc.load_gather(sx0, [iota, zero])
                v1 = plsc.load_gather(sx1, [iota, zero])
                pos = cpos[s]
                m2 = iota < (cnt - g * L)
                fix_group(pos, v0, v1, m2)
                return carry

            lax.fori_loop(NCORR // L, (cnt + L - 1) // L, merge_slow, 0)
            pltpu.sync_copy(o_v, out_hbm.at[pl.ds(slot_off(slot), C)])

        prefill_corr(A)
        prefill_corr(B)
        start_rz(A, 0)
        cntA0 = fire_slot(A)
        start_rz(B, 1)

        def pair_body(h, cntA):
            cntB = fire_slot(B)
            start_rz(A, 2 * h + 2)
            finish_slot(A, 2 * h, cntA)
            cntA_new = fire_slot(A)
            start_rz(B, 2 * h + 3)
            finish_slot(B, 2 * h + 1, cntB)
            return cntA_new

        cntA_last = lax.fori_loop(0, H, pair_body, cntA0)
        finish_slot(A, 2 * H, cntA_last)
        drain_rz(B)

    return k


_sc_interp = _make_kernel()


def kernel(r, z, timetable):
    tab_blocks = timetable.reshape(M, 16)
    return _sc_interp(r, z, tab_blocks)


# speedup vs baseline: 1.0941x; 1.0828x over previous
import functools

import jax
import jax.numpy as jnp
from jax import lax
from jax.experimental import pallas as pl
from jax.experimental.pallas import tpu as pltpu
from jax.experimental.pallas import tpu_sc as plsc

NR = 8192
NZ = 2048
N_QUERY = 1000000

NC = 2
NS = 16
NW = NC * NS
L = 16

C = 1024
GW = 128
NCHUNK = (N_QUERY + C - 1) // C
K = (NCHUNK + NW - 1) // NW
H = (K - 1) // 2


def _make_kernel():
    mesh = plsc.VectorSubcoreMesh(core_axis_name="c", subcore_axis_name="s")

    set_scratch = [
        pltpu.VMEM((C,), jnp.float32),
        pltpu.VMEM((C,), jnp.float32),
        pltpu.VMEM((4 * C,), jnp.int32),
        pltpu.VMEM((4 * C,), jnp.float32),
        pltpu.VMEM((C,), jnp.float32),
        pltpu.VMEM((C,), jnp.float32),
        pltpu.VMEM((C,), jnp.float32),
        pltpu.SemaphoreType.DMA,
        pltpu.SemaphoreType.DMA,
    ]

    @functools.partial(
        pl.kernel,
        mesh=mesh,
        out_type=jax.ShapeDtypeStruct((N_QUERY,), jnp.float32),
        scratch_types=set_scratch + set_scratch,
    )
    def k(r_hbm, z_hbm, tab_hbm, out_hbm,
          rA, zA, iA, tA, wrA, wzA, oA, semzA, semgA,
          rB, zB, iB, tB, wrB, wzB, oB, semzB, semgB):
        wid = lax.axis_index("s") * NC + lax.axis_index("c")
        A = (rA, zA, iA, tA, wrA, wzA, oA, semzA, semgA)
        B = (rB, zB, iB, tB, wrB, wzB, oB, semzB, semgB)

        def slot_off(slot):
            cid = wid + slot * NW
            return jnp.minimum(cid * C, N_QUERY - C)

        def start_rz(bufs, slot):
            r_v, z_v = bufs[0], bufs[1]
            sem = bufs[7]
            off = slot_off(slot)
            pltpu.async_copy(r_hbm.at[pl.ds(off, C)], r_v, sem)
            pltpu.async_copy(z_hbm.at[pl.ds(off, C)], z_v, sem)

        def drain_rz(bufs):
            r_v, z_v = bufs[0], bufs[1]
            sem = bufs[7]
            pltpu.make_async_copy(r_hbm.at[pl.ds(0, C)], r_v, sem).wait()
            pltpu.make_async_copy(z_hbm.at[pl.ds(0, C)], z_v, sem).wait()

        def fire_slot(bufs):
            r_v, z_v, i_v, t_v = bufs[0], bufs[1], bufs[2], bufs[3]
            wr_v, wz_v = bufs[4], bufs[5]
            sem_g = bufs[8]
            drain_rz(bufs)

            def idx_body(i, carry):
                s = pl.ds(i * L, L)
                rr = r_v[s]
                zz = z_v[s]
                ir0 = jnp.minimum(jnp.maximum(rr.astype(jnp.int32), 0), NR - 2)
                iz0 = jnp.minimum(jnp.maximum(zz.astype(jnp.int32), 0), NZ - 2)
                wr = jnp.clip(rr - ir0.astype(jnp.float32), 0.0, 1.0)
                wz = jnp.clip(zz - iz0.astype(jnp.float32), 0.0, 1.0)
                b = ir0 * NZ + iz0
                i_v[pl.ds(0 * C + i * L, L)] = b
                i_v[pl.ds(1 * C + i * L, L)] = b + 1
                i_v[pl.ds(2 * C + i * L, L)] = b + NZ
                i_v[pl.ds(3 * C + i * L, L)] = b + NZ + 1
                wr_v[s] = wr
                wz_v[s] = wz
                return carry

            lax.fori_loop(0, C // L, idx_body, 0)

            for q in range(4):
                for j in range(C // GW):
                    s = pl.ds(q * C + j * GW, GW)
                    pltpu.async_copy(tab_hbm.at[i_v.at[s]], t_v.at[s], sem_g)

        def finish_slot(bufs, slot):
            t_v, wr_v, wz_v, o_v = bufs[3], bufs[4], bufs[5], bufs[6]
            sem_g = bufs[8]
            pltpu.make_async_copy(tab_hbm.at[pl.ds(0, 4 * C)], t_v, sem_g).wait()

            def comb_body(i, carry):
                s = pl.ds(i * L, L)
                wr = wr_v[s]
                wz = wz_v[s]
                t00 = t_v[pl.ds(0 * C + i * L, L)]
                t01 = t_v[pl.ds(1 * C + i * L, L)]
                t10 = t_v[pl.ds(2 * C + i * L, L)]
                t11 = t_v[pl.ds(3 * C + i * L, L)]
                a = t00 * (1.0 - wr) + t10 * wr
                b2 = t01 * (1.0 - wr) + t11 * wr
                o_v[s] = a * (1.0 - wz) + b2 * wz
                return carry

            lax.fori_loop(0, C // L, comb_body, 0)
            pltpu.sync_copy(o_v, out_hbm.at[pl.ds(slot_off(slot), C)])

        start_rz(A, 0)
        fire_slot(A)
        start_rz(B, 1)

        def pair_body(h, carry):
            fire_slot(B)
            start_rz(A, 2 * h + 2)
            finish_slot(A, 2 * h)
            fire_slot(A)
            start_rz(B, 2 * h + 3)
            finish_slot(B, 2 * h + 1)
            return carry

        lax.fori_loop(0, H, pair_body, 0)
        finish_slot(A, 2 * H)
        drain_rz(B)

    return k


_sc_interp = _make_kernel()


def kernel(r, z, timetable):
    tab_flat = timetable.reshape(NR * NZ)
    return _sc_interp(r, z, tab_flat)
